# baseline (speedup 1.0000x reference)
import functools

import jax
import jax.numpy as jnp
from jax import lax
from jax.experimental import pallas as pl
from jax.experimental.pallas import tpu as pltpu
from jax.experimental.pallas import tpu_sc as plsc

N = 64
NUM_TABLES = 7
TBL_ROWS = NUM_TABLES * N + 8
H_BLK = 256
NEG = -1e30
SC_CHUNK = 64


def _map_kernel(x_ref, map_ref):
    xb = x_ref[0]
    hb = xb.shape[0]
    c_io = jax.lax.broadcasted_iota(jnp.int32, (hb, N * N), 1)
    e_io = jnp.bitwise_and(c_io, N - 1)
    s_io = jnp.right_shift(c_io, 6)
    tri = e_io >= s_io
    xt = pltpu.repeat(xb, N, axis=1)
    m = jnp.where(tri, xt, NEG)
    sh = 1
    while sh < N:
        rolled = pltpu.roll(m, sh, axis=1)
        m = jnp.maximum(m, jnp.where(e_io >= sh, rolled, NEG))
        sh *= 2
    map_ref[0] = jnp.where(tri, m, 0.0).reshape(hb, N, N)


def _tables_kernel(xt_ref, tbl_ref):
    xbt = xt_ref[0]
    h = xbt.shape[1]
    tables = [xbt]
    w = xbt
    for k in range(NUM_TABLES - 1):
        step = 1 << k
        w = jnp.maximum(w, pltpu.roll(w, N - step, axis=0))
        tables.append(w)
    tbl_ref[0, :NUM_TABLES * N] = jnp.concatenate(tables, axis=0)
    tbl_ref[0, NUM_TABLES * N:] = jnp.zeros((8, h), jnp.float32)


def _sc_gather_kernel(tbl_ref, i1_ref, i2_ref, out_ref,
                      idx1_v, idx2_v, r1_v, r2_v, sem1, sem2, *,
                      pairs_per_w, num_cores, h):
    wid = lax.axis_index("s") * num_cores + lax.axis_index("c")
    base = wid * pairs_per_w
    for c in range(pairs_per_w // SC_CHUNK):
        off = base + c * SC_CHUNK
        pltpu.sync_copy(i1_ref.at[pl.ds(off, SC_CHUNK)], idx1_v)
        pltpu.sync_copy(i2_ref.at[pl.ds(off, SC_CHUNK)], idx2_v)
        cp1 = pltpu.async_copy(tbl_ref.at[idx1_v], r1_v, sem1)
        cp2 = pltpu.async_copy(tbl_ref.at[idx2_v], r2_v, sem2)
        cp1.wait()
        cp2.wait()

        def body(i, carry):
            for j in range(h // 16):
                sl = pl.ds(j * 16, 16)
                r1_v[i, sl] = jnp.maximum(r1_v[i, sl], r2_v[i, sl])
            return carry

        lax.fori_loop(0, SC_CHUNK, body, 0)
        pltpu.sync_copy(r1_v, out_ref.at[pl.ds(off, SC_CHUNK)])


def kernel(x, props):
    B, H, n = x.shape
    assert n == N
    P = props.shape[0]

    idx0 = props[:, 0].astype(jnp.int32)
    idx1 = (props[:, 1].astype(jnp.int32) - 1) % N
    valid = idx1 >= idx0
    length = idx1 - idx0 + 1
    k = ((length >= 2).astype(jnp.int32) + (length >= 4) + (length >= 8)
         + (length >= 16) + (length >= 32) + (length >= 64))
    row1 = jnp.where(valid, k * N + idx0, NUM_TABLES * N)
    row2 = jnp.where(valid, k * N + (idx1 - (1 << k) + 1), NUM_TABLES * N)
    rbase = jnp.arange(B, dtype=jnp.int32)[:, None] * TBL_ROWS
    r1g = (rbase + row1[None, :]).reshape(-1)
    r2g = (rbase + row2[None, :]).reshape(-1)

    xt = jnp.transpose(x, (0, 2, 1))
    tbl = pl.pallas_call(
        _tables_kernel,
        grid=(B,),
        in_specs=[pl.BlockSpec((1, N, H), lambda b: (b, 0, 0))],
        out_specs=pl.BlockSpec((1, TBL_ROWS, H), lambda b: (b, 0, 0)),
        out_shape=jax.ShapeDtypeStruct((B, TBL_ROWS, H), jnp.float32),
    )(xt)
    tbl_flat = tbl.reshape(B * TBL_ROWS, H)

    info = plsc.get_sparse_core_info()
    num_workers = info.num_cores * info.num_subcores
    pairs_per_w = (B * P) // num_workers
    mesh = plsc.VectorSubcoreMesh(core_axis_name="c", subcore_axis_name="s")
    props_flat = pl.kernel(
        functools.partial(_sc_gather_kernel, pairs_per_w=pairs_per_w,
                          num_cores=info.num_cores, h=H),
        out_type=jax.ShapeDtypeStruct((B * P, H), jnp.float32),
        mesh=mesh,
        scratch_types=[
            pltpu.VMEM((SC_CHUNK,), jnp.int32),
            pltpu.VMEM((SC_CHUNK,), jnp.int32),
            pltpu.VMEM((SC_CHUNK, H), jnp.float32),
            pltpu.VMEM((SC_CHUNK, H), jnp.float32),
            pltpu.SemaphoreType.DMA,
            pltpu.SemaphoreType.DMA,
        ],
    )(tbl_flat, r1g, r2g)
    props_h = props_flat.reshape(B, P, H)

    map_h = pl.pallas_call(
        _map_kernel,
        grid=(B, H // H_BLK),
        in_specs=[pl.BlockSpec((1, H_BLK, N), lambda b, hh: (b, hh, 0))],
        out_specs=pl.BlockSpec((1, H_BLK, N, N), lambda b, hh: (b, hh, 0, 0)),
        out_shape=jax.ShapeDtypeStruct((B, H, N, N), jnp.float32),
    )(x)

    tri = (jnp.arange(N)[:, None] <= jnp.arange(N)[None, :]).astype(x.dtype)
    map_mask = jnp.broadcast_to(tri[None, None], (B, 1, N, N))
    return props_h, map_h, map_mask


# speedup vs baseline: 2.3547x; 2.3547x over previous
import functools

import jax
import jax.numpy as jnp
from jax import lax
from jax.experimental import pallas as pl
from jax.experimental.pallas import tpu as pltpu
from jax.experimental.pallas import tpu_sc as plsc

N = 64
NUM_TABLES = 7
TBL_ROWS = NUM_TABLES * N + 8
H_BLK = 256
NEG = -1e30
SC_CHUNK = 64


def _map_kernel(xt_ref, map_ref):
    xbt = xt_ref[0]
    hb = xbt.shape[1]
    s_io = jax.lax.broadcasted_iota(jnp.int32, (N, N, hb), 0)
    e_io = jax.lax.broadcasted_iota(jnp.int32, (N, N, hb), 1)
    tri = e_io >= s_io
    m = jnp.where(tri, xbt[None, :, :], NEG)
    sh = 1
    while sh < N:
        rolled = pltpu.roll(m, sh, axis=1)
        m = jnp.maximum(m, jnp.where(e_io >= sh, rolled, NEG))
        sh *= 2
    map_ref[0] = jnp.where(tri, m, 0.0)


def _tables_kernel(xt_ref, tbl_ref):
    xbt = xt_ref[0]
    h = xbt.shape[1]
    tables = [xbt]
    w = xbt
    for k in range(NUM_TABLES - 1):
        step = 1 << k
        w = jnp.maximum(w, pltpu.roll(w, N - step, axis=0))
        tables.append(w)
    tbl_ref[0, :NUM_TABLES * N] = jnp.concatenate(tables, axis=0)
    tbl_ref[0, NUM_TABLES * N:] = jnp.zeros((8, h), jnp.float32)


def _sc_gather_kernel(tbl_ref, i1_ref, i2_ref, out_ref,
                      idx1_v, idx2_v, r1_v, r2_v, sem1, sem2, *,
                      pairs_per_w, num_cores, h):
    wid = lax.axis_index("s") * num_cores + lax.axis_index("c")
    base = wid * pairs_per_w
    for c in range(pairs_per_w // SC_CHUNK):
        off = base + c * SC_CHUNK
        pltpu.sync_copy(i1_ref.at[pl.ds(off, SC_CHUNK)], idx1_v)
        pltpu.sync_copy(i2_ref.at[pl.ds(off, SC_CHUNK)], idx2_v)
        cp1 = pltpu.async_copy(tbl_ref.at[idx1_v], r1_v, sem1)
        cp2 = pltpu.async_copy(tbl_ref.at[idx2_v], r2_v, sem2)
        cp1.wait()
        cp2.wait()

        def body(i, carry):
            for j in range(h // 16):
                sl = pl.ds(j * 16, 16)
                r1_v[i, sl] = jnp.maximum(r1_v[i, sl], r2_v[i, sl])
            return carry

        lax.fori_loop(0, SC_CHUNK, body, 0)
        pltpu.sync_copy(r1_v, out_ref.at[pl.ds(off, SC_CHUNK)])


def kernel(x, props):
    B, H, n = x.shape
    assert n == N
    P = props.shape[0]

    idx0 = props[:, 0].astype(jnp.int32)
    idx1 = (props[:, 1].astype(jnp.int32) - 1) % N
    valid = idx1 >= idx0
    length = idx1 - idx0 + 1
    k = ((length >= 2).astype(jnp.int32) + (length >= 4) + (length >= 8)
         + (length >= 16) + (length >= 32) + (length >= 64))
    row1 = jnp.where(valid, k * N + idx0, NUM_TABLES * N)
    row2 = jnp.where(valid, k * N + (idx1 - (1 << k) + 1), NUM_TABLES * N)
    rbase = jnp.arange(B, dtype=jnp.int32)[:, None] * TBL_ROWS
    r1g = (rbase + row1[None, :]).reshape(-1)
    r2g = (rbase + row2[None, :]).reshape(-1)

    xt = jnp.transpose(x, (0, 2, 1))
    tbl = pl.pallas_call(
        _tables_kernel,
        grid=(B,),
        in_specs=[pl.BlockSpec((1, N, H), lambda b: (b, 0, 0))],
        out_specs=pl.BlockSpec((1, TBL_ROWS, H), lambda b: (b, 0, 0)),
        out_shape=jax.ShapeDtypeStruct((B, TBL_ROWS, H), jnp.float32),
    )(xt)
    tbl_flat = tbl.reshape(B * TBL_ROWS, H)

    info = plsc.get_sparse_core_info()
    num_workers = info.num_cores * info.num_subcores
    pairs_per_w = (B * P) // num_workers
    mesh = plsc.VectorSubcoreMesh(core_axis_name="c", subcore_axis_name="s")
    props_flat = pl.kernel(
        functools.partial(_sc_gather_kernel, pairs_per_w=pairs_per_w,
                          num_cores=info.num_cores, h=H),
        out_type=jax.ShapeDtypeStruct((B * P, H), jnp.float32),
        mesh=mesh,
        scratch_types=[
            pltpu.VMEM((SC_CHUNK,), jnp.int32),
            pltpu.VMEM((SC_CHUNK,), jnp.int32),
            pltpu.VMEM((SC_CHUNK, H), jnp.float32),
            pltpu.VMEM((SC_CHUNK, H), jnp.float32),
            pltpu.SemaphoreType.DMA,
            pltpu.SemaphoreType.DMA,
        ],
    )(tbl_flat, r1g, r2g)
    props_h = props_flat.reshape(B, P, H)

    map_t = pl.pallas_call(
        _map_kernel,
        grid=(B, H // H_BLK),
        in_specs=[pl.BlockSpec((1, N, H_BLK), lambda b, hh: (b, 0, hh))],
        out_specs=pl.BlockSpec((1, N, N, H_BLK), lambda b, hh: (b, 0, 0, hh)),
        out_shape=jax.ShapeDtypeStruct((B, N, N, H), jnp.float32),
    )(xt)
    map_h = jnp.transpose(map_t, (0, 3, 1, 2))

    tri = (jnp.arange(N)[:, None] <= jnp.arange(N)[None, :]).astype(x.dtype)
    map_mask = jnp.broadcast_to(tri[None, None], (B, 1, N, N))
    return props_h, map_h, map_mask
